# baseline (speedup 1.0000x reference)
import functools

import jax
import jax.numpy as jnp
from jax import lax
from jax.experimental import pallas as pl
from jax.experimental.pallas import tpu as pltpu
from jax.experimental.pallas import tpu_sc as plsc

H = 128
NREG = H // 16
EPS = 1e-12

NC = 2
NS = 16
NW = NC * NS

C = 128


def _rsqrt16(x):
  i = lax.bitcast_convert_type(x, jnp.int32)
  i = jnp.int32(0x5F3759DF) - lax.shift_right_arithmetic(i, 1)
  y = lax.bitcast_convert_type(i, jnp.float32)
  for _ in range(3):
    y = y * (1.5 - 0.5 * x * y * y)
  return y


def _sc_body(wi, ai, si, pi, yi, Ww, Ws, Wa, Wy, Wp, g, b, out,
             iw, ia, is_, ip, iy, rw, rs, ra, ry, rp, gv, bv, sem,
             *, per_worker):
  cid = lax.axis_index("c")
  sid = lax.axis_index("s")
  wid = sid * NC + cid

  pltpu.sync_copy(g, gv)
  pltpu.sync_copy(b, bv)
  gregs = [gv[pl.ds(16 * j, 16)] for j in range(NREG)]
  bregs = [bv[pl.ds(16 * j, 16)] for j in range(NREG)]

  nchunks = per_worker // C

  def chunk_body(ci, carry):
    base = wid * per_worker + ci * C

    pltpu.sync_copy(wi.at[pl.ds(base, C)], iw)
    pltpu.sync_copy(ai.at[pl.ds(base, C)], ia)
    pltpu.sync_copy(si.at[pl.ds(base, C)], is_)
    pltpu.sync_copy(pi.at[pl.ds(base, C)], ip)
    pltpu.sync_copy(yi.at[pl.ds(base, C)], iy)

    h0 = pltpu.async_copy(Ww.at[iw], rw, sem)
    h1 = pltpu.async_copy(Ws.at[is_], rs, sem)
    h2 = pltpu.async_copy(Wa.at[ia], ra, sem)
    h3 = pltpu.async_copy(Wy.at[iy], ry, sem)
    h4 = pltpu.async_copy(Wp.at[ip], rp, sem)
    h0.wait()
    h1.wait()
    h2.wait()
    h3.wait()
    h4.wait()

    def tok(t, tc):
      vs = [rw[t, pl.ds(16 * j, 16)] + rs[t, pl.ds(16 * j, 16)]
            + ra[t, pl.ds(16 * j, 16)] + ry[t, pl.ds(16 * j, 16)]
            + rp[t, pl.ds(16 * j, 16)] for j in range(NREG)]
      tot = vs[0] + vs[1]
      for j in range(2, NREG):
        tot = tot + vs[j]
      mean = jnp.broadcast_to(jnp.sum(tot) * (1.0 / H), (16,))
      ds = [v - mean for v in vs]
      q = ds[0] * ds[0] + ds[1] * ds[1]
      for j in range(2, NREG):
        q = q + ds[j] * ds[j]
      var = jnp.broadcast_to(jnp.sum(q) * (1.0 / H) + EPS, (16,))
      rstd = _rsqrt16(var)
      for j in range(NREG):
        rw[t, pl.ds(16 * j, 16)] = ds[j] * rstd * gregs[j] + bregs[j]
      return tc

    lax.fori_loop(0, C, tok, 0)
    pltpu.sync_copy(rw, out.at[pl.ds(base, C)])
    return carry

  lax.fori_loop(0, nchunks, chunk_body, 0)


def kernel(word_ids, age_ids, seg_ids, posi_ids, year_ids,
           W_word, W_seg, W_age, W_year, posi_table, ln_gamma, ln_beta):
  B, L = word_ids.shape
  N = B * L
  per_worker = N // NW

  wi = word_ids.reshape(N).astype(jnp.int32)
  ai = age_ids.reshape(N).astype(jnp.int32)
  si = seg_ids.reshape(N).astype(jnp.int32)
  pi = posi_ids.reshape(N).astype(jnp.int32)
  yi = year_ids.reshape(N).astype(jnp.int32)

  mesh = plsc.VectorSubcoreMesh(core_axis_name="c", subcore_axis_name="s",
                                num_cores=NC, num_subcores=NS)
  body = functools.partial(_sc_body, per_worker=per_worker)
  out = pl.kernel(
      body,
      out_type=jax.ShapeDtypeStruct((N, H), jnp.float32),
      mesh=mesh,
      compiler_params=pltpu.CompilerParams(needs_layout_passes=False),
      scratch_types=[
          pltpu.VMEM((C,), jnp.int32),
          pltpu.VMEM((C,), jnp.int32),
          pltpu.VMEM((C,), jnp.int32),
          pltpu.VMEM((C,), jnp.int32),
          pltpu.VMEM((C,), jnp.int32),
          pltpu.VMEM((C, H), jnp.float32),
          pltpu.VMEM((C, H), jnp.float32),
          pltpu.VMEM((C, H), jnp.float32),
          pltpu.VMEM((C, H), jnp.float32),
          pltpu.VMEM((C, H), jnp.float32),
          pltpu.VMEM((H,), jnp.float32),
          pltpu.VMEM((H,), jnp.float32),
          pltpu.SemaphoreType.DMA,
      ],
  )(wi, ai, si, pi, yi, W_word, W_seg, W_age, W_year, posi_table,
    ln_gamma, ln_beta)
  return out.reshape(B, L, H)


# speedup vs baseline: 1.0148x; 1.0148x over previous
import functools

import jax
import jax.numpy as jnp
from jax import lax
from jax.experimental import pallas as pl
from jax.experimental.pallas import tpu as pltpu
from jax.experimental.pallas import tpu_sc as plsc

H = 128
NREG = H // 16
EPS = 1e-12

NC = 2
NS = 16
NW = NC * NS

C = 128


def _rsqrt16(x):
  i = lax.bitcast_convert_type(x, jnp.int32)
  i = jnp.int32(0x5F3759DF) - lax.shift_right_arithmetic(i, 1)
  y = lax.bitcast_convert_type(i, jnp.float32)
  for _ in range(3):
    y = y * (1.5 - 0.5 * x * y * y)
  return y


def _sc_body(wi, ai, si, pi, yi, Ww, Ws, Wa, Wy, Wp, g, b, out,
             iw, ia, is_, ip, iy, rw, rs, ra, ry, rp, ro, gv, bv, sem,
             *, per_worker):
  cid = lax.axis_index("c")
  sid = lax.axis_index("s")
  wid = sid * NC + cid

  pltpu.sync_copy(g, gv)
  pltpu.sync_copy(b, bv)
  gregs = [gv[pl.ds(16 * j, 16)] for j in range(NREG)]
  bregs = [bv[pl.ds(16 * j, 16)] for j in range(NREG)]

  nchunks = per_worker // C

  def chunk_body(ci, carry):
    base = wid * per_worker + ci * C

    pltpu.sync_copy(wi.at[pl.ds(base, C)], iw)
    pltpu.sync_copy(ai.at[pl.ds(base, C)], ia)
    pltpu.sync_copy(si.at[pl.ds(base, C)], is_)
    pltpu.sync_copy(pi.at[pl.ds(base, C)], ip)
    pltpu.sync_copy(yi.at[pl.ds(base, C)], iy)

    h0 = pltpu.async_copy(Ww.at[iw], rw, sem)
    h1 = pltpu.async_copy(Ws.at[is_], rs, sem)
    h2 = pltpu.async_copy(Wa.at[ia], ra, sem)
    h3 = pltpu.async_copy(Wy.at[iy], ry, sem)
    h4 = pltpu.async_copy(Wp.at[ip], rp, sem)
    h0.wait()
    h1.wait()
    h2.wait()
    h3.wait()
    h4.wait()

    @plsc.parallel_loop(0, C, unroll=4)
    def tok(t):
      vs = [rw[t, pl.ds(16 * j, 16)] + rs[t, pl.ds(16 * j, 16)]
            + ra[t, pl.ds(16 * j, 16)] + ry[t, pl.ds(16 * j, 16)]
            + rp[t, pl.ds(16 * j, 16)] for j in range(NREG)]
      tot = vs[0] + vs[1]
      for j in range(2, NREG):
        tot = tot + vs[j]
      mean = jnp.broadcast_to(jnp.sum(tot) * (1.0 / H), (16,))
      ds = [v - mean for v in vs]
      q = ds[0] * ds[0] + ds[1] * ds[1]
      for j in range(2, NREG):
        q = q + ds[j] * ds[j]
      var = jnp.broadcast_to(jnp.sum(q) * (1.0 / H) + EPS, (16,))
      rstd = _rsqrt16(var)
      for j in range(NREG):
        ro[t, pl.ds(16 * j, 16)] = ds[j] * rstd * gregs[j] + bregs[j]

    pltpu.sync_copy(ro, out.at[pl.ds(base, C)])
    return carry

  lax.fori_loop(0, nchunks, chunk_body, 0)


def kernel(word_ids, age_ids, seg_ids, posi_ids, year_ids,
           W_word, W_seg, W_age, W_year, posi_table, ln_gamma, ln_beta):
  B, L = word_ids.shape
  N = B * L
  per_worker = N // NW

  wi = word_ids.reshape(N).astype(jnp.int32)
  ai = age_ids.reshape(N).astype(jnp.int32)
  si = seg_ids.reshape(N).astype(jnp.int32)
  pi = posi_ids.reshape(N).astype(jnp.int32)
  yi = year_ids.reshape(N).astype(jnp.int32)

  mesh = plsc.VectorSubcoreMesh(core_axis_name="c", subcore_axis_name="s",
                                num_cores=NC, num_subcores=NS)
  body = functools.partial(_sc_body, per_worker=per_worker)
  out = pl.kernel(
      body,
      out_type=jax.ShapeDtypeStruct((N, H), jnp.float32),
      mesh=mesh,
      compiler_params=pltpu.CompilerParams(needs_layout_passes=False),
      scratch_types=[
          pltpu.VMEM((C,), jnp.int32),
          pltpu.VMEM((C,), jnp.int32),
          pltpu.VMEM((C,), jnp.int32),
          pltpu.VMEM((C,), jnp.int32),
          pltpu.VMEM((C,), jnp.int32),
          pltpu.VMEM((C, H), jnp.float32),
          pltpu.VMEM((C, H), jnp.float32),
          pltpu.VMEM((C, H), jnp.float32),
          pltpu.VMEM((C, H), jnp.float32),
          pltpu.VMEM((C, H), jnp.float32),
          pltpu.VMEM((C, H), jnp.float32),
          pltpu.VMEM((H,), jnp.float32),
          pltpu.VMEM((H,), jnp.float32),
          pltpu.SemaphoreType.DMA,
      ],
  )(wi, ai, si, pi, yi, W_word, W_seg, W_age, W_year, posi_table,
    ln_gamma, ln_beta)
  return out.reshape(B, L, H)
